# baseline (speedup 1.0000x reference)
import functools

import jax
import jax.numpy as jnp
from jax import lax
from jax.experimental import pallas as pl
from jax.experimental.pallas import tpu as pltpu
from jax.experimental.pallas import tpu_sc as plsc

POOL = 7
PQ = POOL * POOL
LANES = 16
NC, NS = 2, 16
NW = NC * NS
NJ = (PQ + 1) // 2


def _take_lane(vec, i):
    return jnp.take_along_axis(vec, jnp.full((LANES,), i, jnp.int32), axis=0)


@functools.partial(jax.jit, static_argnums=(1, 2, 3))
def _roi_pool_sc(args, N, HW, C):
    H, W = HW
    nchunks = N // LANES
    tpw = -(-nchunks // NW)
    QPAD = 8
    nrows = N * POOL * QPAD
    CV = C // LANES

    def body(table, ry1, rx1, ry2, rx2, out, roi_v, rows0_v, rows1_v, cols0_v, cols1_v,
             rf_v, cf_v, idx0_v, idx1_v, sidx_v, gbuf0_v, gbuf1_v, outp_v,
             gsem0, gsem1):
        wid = lax.axis_index("s") * NC + lax.axis_index("c")

        def chunk_body(t, carry):
            chunk = wid + t * NW

            @pl.when(chunk < nchunks)
            def _():
                base = chunk * LANES
                pltpu.sync_copy(ry1.at[pl.ds(base, LANES)], roi_v.at[0])
                pltpu.sync_copy(rx1.at[pl.ds(base, LANES)], roi_v.at[1])
                pltpu.sync_copy(ry2.at[pl.ds(base, LANES)], roi_v.at[2])
                pltpu.sync_copy(rx2.at[pl.ds(base, LANES)], roi_v.at[3])
                y1 = roi_v[0, :]
                x1 = roi_v[1, :]
                y2 = roi_v[2, :]
                x2 = roi_v[3, :]
                h = jnp.maximum(x2 - x1, 1)
                w = jnp.maximum(y2 - y1, 1)
                hstep = h.astype(jnp.float32) / float(POOL)
                wstep = w.astype(jnp.float32) / float(POOL)
                for p in range(POOL):
                    rpos = jnp.float32(p) * hstep
                    r0 = rpos.astype(jnp.int32)
                    r1 = jnp.minimum(r0 + 1, h - 1)
                    rows0_v[p, :] = jnp.clip(x1 + r0, 0, H - 1) * W
                    rows1_v[p, :] = jnp.clip(x1 + r1, 0, H - 1) * W
                    rf_v[p, :] = rpos - r0.astype(jnp.float32)
                    cpos = jnp.float32(p) * wstep
                    c0 = cpos.astype(jnp.int32)
                    c1 = jnp.minimum(c0 + 1, w - 1)
                    cols0_v[p, :] = jnp.clip(y1 + c0, 0, W - 1)
                    cols1_v[p, :] = jnp.clip(y1 + c1, 0, W - 1)
                    cf_v[p, :] = cpos - c0.astype(jnp.float32)
                rbase = (base + lax.iota(jnp.int32, LANES)) * (POOL * QPAD)

                def build_idx(j, idx_ref):
                    for s in range(2):
                        pq = jnp.minimum(2 * j + s, PQ - 1)
                        p = pq // POOL
                        q = pq % POOL
                        r0w = rows0_v[p, :]
                        r1w = rows1_v[p, :]
                        c0 = cols0_v[q, :]
                        c1 = cols1_v[q, :]
                        off = s * 4 * LANES
                        idx_ref[pl.ds(off, LANES)] = r0w + c0
                        idx_ref[pl.ds(off + LANES, LANES)] = r0w + c1
                        idx_ref[pl.ds(off + 2 * LANES, LANES)] = r1w + c0
                        idx_ref[pl.ds(off + 3 * LANES, LANES)] = r1w + c1

                def fire(idx_ref, gbuf_ref, sem):
                    pltpu.async_copy(table.at[idx_ref], gbuf_ref, sem)

                def drain(idx_ref, gbuf_ref, sem):
                    pltpu.make_async_copy(table.at[idx_ref], gbuf_ref, sem).wait()

                def weights(p, q):
                    rfp = rf_v[p, :]
                    cfq = cf_v[q, :]
                    return ((1.0 - rfp) * (1.0 - cfq), (1.0 - rfp) * cfq,
                            rfp * (1.0 - cfq), rfp * cfq)

                def compute2(pq_a, pq_b, gbuf_ref):
                    p_a, q_a = pq_a // POOL, pq_a % POOL
                    p_b, q_b = pq_b // POOL, pq_b % POOL
                    par_a = lax.rem(p_a, 2)
                    par_b = lax.rem(p_b, 2)
                    wa = weights(p_a, q_a)
                    wb = weights(p_b, q_b)
                    sidx_v[par_a, pl.ds(q_a * LANES, LANES)] = (
                        rbase + (p_a * QPAD + q_a))
                    sidx_v[par_b, pl.ds(q_b * LANES, LANES)] = (
                        rbase + (p_b * QPAD + q_b))

                    @plsc.parallel_loop(0, LANES, 1, unroll=2)
                    def i_body(i):
                        wa00 = _take_lane(wa[0], i)
                        wa01 = _take_lane(wa[1], i)
                        wa10 = _take_lane(wa[2], i)
                        wa11 = _take_lane(wa[3], i)
                        wb00 = _take_lane(wb[0], i)
                        wb01 = _take_lane(wb[1], i)
                        wb10 = _take_lane(wb[2], i)
                        wb11 = _take_lane(wb[3], i)
                        row_a = q_a * LANES + i
                        row_b = q_b * LANES + i
                        for cc in range(CV):
                            sl = pl.ds(cc * LANES, LANES)
                            outp_v[par_a, row_a, sl] = (
                                wa00 * gbuf_ref[i, sl]
                                + wa01 * gbuf_ref[LANES + i, sl]
                                + wa10 * gbuf_ref[2 * LANES + i, sl]
                                + wa11 * gbuf_ref[3 * LANES + i, sl])
                            outp_v[par_b, row_b, sl] = (
                                wb00 * gbuf_ref[4 * LANES + i, sl]
                                + wb01 * gbuf_ref[5 * LANES + i, sl]
                                + wb10 * gbuf_ref[6 * LANES + i, sl]
                                + wb11 * gbuf_ref[7 * LANES + i, sl])

                    @pl.when(q_a == POOL - 1)
                    def _():
                        pltpu.sync_copy(outp_v.at[par_a], out.at[sidx_v.at[par_a]])

                    @pl.when((q_b == POOL - 1) & (pq_b != pq_a))
                    def _():
                        pltpu.sync_copy(outp_v.at[par_b], out.at[sidx_v.at[par_b]])

                build_idx(0, idx0_v)
                fire(idx0_v, gbuf0_v, gsem0)
                build_idx(1, idx1_v)
                fire(idx1_v, gbuf1_v, gsem1)

                def jj_body(jj, carry_j):
                    drain(idx0_v, gbuf0_v, gsem0)
                    compute2(4 * jj, jnp.minimum(4 * jj + 1, PQ - 1), gbuf0_v)

                    @pl.when(2 * jj + 2 < NJ)
                    def _():
                        build_idx(2 * jj + 2, idx0_v)
                        fire(idx0_v, gbuf0_v, gsem0)

                    @pl.when(2 * jj + 1 < NJ)
                    def _():
                        drain(idx1_v, gbuf1_v, gsem1)
                        compute2(4 * jj + 2, 4 * jj + 3, gbuf1_v)

                        @pl.when(2 * jj + 3 < NJ)
                        def _():
                            build_idx(2 * jj + 3, idx1_v)
                            fire(idx1_v, gbuf1_v, gsem1)

                    return carry_j

                lax.fori_loop(0, (NJ + 1) // 2, jj_body, 0)

            return carry

        lax.fori_loop(0, tpw, chunk_body, 0)

    call = pl.kernel(
        body,
        out_type=jax.ShapeDtypeStruct((nrows, C), jnp.float32),
        mesh=plsc.VectorSubcoreMesh(core_axis_name="c", subcore_axis_name="s"),
        scratch_types=[
            pltpu.VMEM((4, LANES), jnp.int32),
            pltpu.VMEM((POOL, LANES), jnp.int32),
            pltpu.VMEM((POOL, LANES), jnp.int32),
            pltpu.VMEM((POOL, LANES), jnp.int32),
            pltpu.VMEM((POOL, LANES), jnp.int32),
            pltpu.VMEM((POOL, LANES), jnp.float32),
            pltpu.VMEM((POOL, LANES), jnp.float32),
            pltpu.VMEM((8 * LANES,), jnp.int32),
            pltpu.VMEM((8 * LANES,), jnp.int32),
            pltpu.VMEM((2, POOL * LANES), jnp.int32),
            pltpu.VMEM((8 * LANES, C), jnp.float32),
            pltpu.VMEM((8 * LANES, C), jnp.float32),
            pltpu.VMEM((2, POOL * LANES, C), jnp.float32),
            pltpu.SemaphoreType.DMA,
            pltpu.SemaphoreType.DMA,
        ],
    )
    return call(*args)


def kernel(features, roi):
    B, H, W, C = features.shape
    N = roi.shape[1]
    table = features.reshape(B * H * W, C)
    r32 = roi[0].astype(jnp.int32)
    coords = tuple(r32[:, j] for j in range(4))
    out = _roi_pool_sc((table,) + coords, N, (H, W), C)
    scale = jnp.float32(1.0) + jnp.float32(0.0) * features[0, 0, 0, 0]
    return out.reshape(N, POOL, 8, C)[:, :, :POOL, :] * scale


# speedup vs baseline: 1.2450x; 1.2450x over previous
import functools

import jax
import jax.numpy as jnp
from jax import lax
from jax.experimental import pallas as pl
from jax.experimental.pallas import tpu as pltpu
from jax.experimental.pallas import tpu_sc as plsc

POOL = 7
PQ = POOL * POOL
LANES = 16
NC, NS = 2, 16
NW = NC * NS
NJ = (PQ + 1) // 2


def _take_lane(vec, i):
    return jnp.take_along_axis(vec, jnp.full((LANES,), i, jnp.int32), axis=0)


@functools.partial(jax.jit, static_argnums=(1, 2, 3))
def _roi_pool_sc(args, N, HW, C):
    H, W = HW
    nchunks = N // LANES
    tpw = -(-nchunks // NW)
    QPAD = 8
    nrows = N * POOL * QPAD
    CV = C // LANES

    def body(table, roit, out, roi_v, rows0_v, rows1_v, cols0_v, cols1_v,
             rf_v, cf_v, idx0_v, idx1_v, sidx_v, gbuf0_v, gbuf1_v, outp_v,
             gsem0, gsem1):
        wid = lax.axis_index("s") * NC + lax.axis_index("c")
        pltpu.sync_copy(roit, roi_v)

        def chunk_body(t, carry):
            chunk = wid + t * NW

            @pl.when(chunk < nchunks)
            def _():
                base = chunk * LANES
                y1 = roi_v[0, pl.ds(base, LANES)]
                x1 = roi_v[1, pl.ds(base, LANES)]
                y2 = roi_v[2, pl.ds(base, LANES)]
                x2 = roi_v[3, pl.ds(base, LANES)]
                h = jnp.maximum(x2 - x1, 1)
                w = jnp.maximum(y2 - y1, 1)
                hstep = h.astype(jnp.float32) / float(POOL)
                wstep = w.astype(jnp.float32) / float(POOL)
                for p in range(POOL):
                    rpos = jnp.float32(p) * hstep
                    r0 = rpos.astype(jnp.int32)
                    r1 = jnp.minimum(r0 + 1, h - 1)
                    rows0_v[p, :] = jnp.clip(x1 + r0, 0, H - 1) * W
                    rows1_v[p, :] = jnp.clip(x1 + r1, 0, H - 1) * W
                    rf_v[p, :] = rpos - r0.astype(jnp.float32)
                    cpos = jnp.float32(p) * wstep
                    c0 = cpos.astype(jnp.int32)
                    c1 = jnp.minimum(c0 + 1, w - 1)
                    cols0_v[p, :] = jnp.clip(y1 + c0, 0, W - 1)
                    cols1_v[p, :] = jnp.clip(y1 + c1, 0, W - 1)
                    cf_v[p, :] = cpos - c0.astype(jnp.float32)
                rbase = (base + lax.iota(jnp.int32, LANES)) * (POOL * QPAD)

                def build_idx(j, idx_ref):
                    for s in range(2):
                        pq = jnp.minimum(2 * j + s, PQ - 1)
                        p = pq // POOL
                        q = pq % POOL
                        r0w = rows0_v[p, :]
                        r1w = rows1_v[p, :]
                        c0 = cols0_v[q, :]
                        c1 = cols1_v[q, :]
                        off = s * 4 * LANES
                        idx_ref[pl.ds(off, LANES)] = r0w + c0
                        idx_ref[pl.ds(off + LANES, LANES)] = r0w + c1
                        idx_ref[pl.ds(off + 2 * LANES, LANES)] = r1w + c0
                        idx_ref[pl.ds(off + 3 * LANES, LANES)] = r1w + c1

                def fire(idx_ref, gbuf_ref, sem):
                    pltpu.async_copy(table.at[idx_ref], gbuf_ref, sem)

                def drain(idx_ref, gbuf_ref, sem):
                    pltpu.make_async_copy(table.at[idx_ref], gbuf_ref, sem).wait()

                def compute(pq, gbuf_ref, srow):
                    p = pq // POOL
                    q = pq % POOL
                    rfp = rf_v[p, :]
                    cfq = cf_v[q, :]
                    w00v = (1.0 - rfp) * (1.0 - cfq)
                    w01v = (1.0 - rfp) * cfq
                    w10v = rfp * (1.0 - cfq)
                    w11v = rfp * cfq
                    sidx_v[pl.ds(q * LANES, LANES)] = rbase + (p * QPAD + q)

                    @plsc.parallel_loop(0, LANES, 1, unroll=2)
                    def i_body(i):
                        w00 = _take_lane(w00v, i)
                        w01 = _take_lane(w01v, i)
                        w10 = _take_lane(w10v, i)
                        w11 = _take_lane(w11v, i)
                        row = q * LANES + i
                        for cc in range(CV):
                            sl = pl.ds(cc * LANES, LANES)
                            outp_v[row, sl] = (
                                w00 * gbuf_ref[srow + i, sl]
                                + w01 * gbuf_ref[srow + LANES + i, sl]
                                + w10 * gbuf_ref[srow + 2 * LANES + i, sl]
                                + w11 * gbuf_ref[srow + 3 * LANES + i, sl])

                    @pl.when(q == POOL - 1)
                    def _():
                        pltpu.sync_copy(outp_v, out.at[sidx_v])

                build_idx(0, idx0_v)
                fire(idx0_v, gbuf0_v, gsem0)
                build_idx(1, idx1_v)
                fire(idx1_v, gbuf1_v, gsem1)

                def jj_body(jj, carry_j):
                    jb = 2 * jj + 1
                    drain(idx0_v, gbuf0_v, gsem0)
                    compute(4 * jj, gbuf0_v, 0)
                    compute(4 * jj + 1, gbuf0_v, 4 * LANES)
                    build_idx(2 * jj + 2, idx0_v)
                    fire(idx0_v, gbuf0_v, gsem0)
                    drain(idx1_v, gbuf1_v, gsem1)
                    compute(4 * jj + 2, gbuf1_v, 0)
                    compute(4 * jj + 3, gbuf1_v, 4 * LANES)

                    @pl.when(jb + 2 < NJ)
                    def _():
                        build_idx(jb + 2, idx1_v)
                        fire(idx1_v, gbuf1_v, gsem1)

                    return carry_j

                lax.fori_loop(0, (NJ - 1) // 2, jj_body, 0)
                drain(idx0_v, gbuf0_v, gsem0)
                compute(PQ - 1, gbuf0_v, 0)

            return carry

        lax.fori_loop(0, tpw, chunk_body, 0)

    call = pl.kernel(
        body,
        out_type=jax.ShapeDtypeStruct((nrows, C), jnp.float32),
        mesh=plsc.VectorSubcoreMesh(core_axis_name="c", subcore_axis_name="s"),
        scratch_types=[
            pltpu.VMEM((4, N), jnp.int32),
            pltpu.VMEM((POOL, LANES), jnp.int32),
            pltpu.VMEM((POOL, LANES), jnp.int32),
            pltpu.VMEM((POOL, LANES), jnp.int32),
            pltpu.VMEM((POOL, LANES), jnp.int32),
            pltpu.VMEM((POOL, LANES), jnp.float32),
            pltpu.VMEM((POOL, LANES), jnp.float32),
            pltpu.VMEM((8 * LANES,), jnp.int32),
            pltpu.VMEM((8 * LANES,), jnp.int32),
            pltpu.VMEM((POOL * LANES,), jnp.int32),
            pltpu.VMEM((8 * LANES, C), jnp.float32),
            pltpu.VMEM((8 * LANES, C), jnp.float32),
            pltpu.VMEM((POOL * LANES, C), jnp.float32),
            pltpu.SemaphoreType.DMA,
            pltpu.SemaphoreType.DMA,
        ],
    )
    return call(*args)


def kernel(features, roi):
    B, H, W, C = features.shape
    N = roi.shape[1]
    table = features.reshape(B * H * W, C)
    roit = roi[0].astype(jnp.int32).T
    out = _roi_pool_sc((table, roit), N, (H, W), C)
    scale = jnp.float32(1.0) + jnp.float32(0.0) * features[0, 0, 0, 0]
    return out.reshape(N, POOL, 8, C)[:, :, :POOL, :] * scale
